# baseline (speedup 1.0000x reference)
import functools

import jax
import jax.numpy as jnp
from jax import lax
from jax.experimental import pallas as pl
from jax.experimental.pallas import tpu as pltpu
from jax.experimental.pallas import tpu_sc as plsc

N = 10000
E = 324000
K = 27
EK = E // K
INC = 128
OUTC = 128

NC = 2
NS = 16
NW = NC * NS

C = 128
NBUF = 2

ROWS_PER_TILE = 632
ACC_R = NS * ROWS_PER_TILE

NB = 5
BLK = N // NB

KA = 7
KB = K - KA
NCH0A, NCH1A = 42, 2
NCH0B, NCH1B = 84, 34


def _matmul_body(x_ref, w_ref, y_ref):
    y_ref[...] = jnp.dot(
        x_ref[...], w_ref[0], preferred_element_type=jnp.float32
    )[None]


def _tc_matmul(x, Wp):
    kspan = Wp.shape[0]
    return pl.pallas_call(
        _matmul_body,
        grid=(NB, kspan),
        in_specs=[
            pl.BlockSpec((BLK, INC), lambda nb, k: (nb, 0)),
            pl.BlockSpec((1, INC, OUTC), lambda nb, k: (k, 0, 0)),
        ],
        out_specs=pl.BlockSpec((1, BLK, OUTC), lambda nb, k: (k, nb, 0)),
        out_shape=jax.ShapeDtypeStruct((kspan, N, OUTC), jnp.float32),
    )(x, Wp).reshape(kspan * N, OUTC)


def _sc_accum(y2, eidx, init, nch0, nch1):
    full_init = init.ndim == 3
    mesh = plsc.VectorSubcoreMesh(
        core_axis_name="c", subcore_axis_name="s", num_cores=NC, num_subcores=NS
    )

    @functools.partial(
        pl.kernel,
        out_type=jax.ShapeDtypeStruct((NC, ACC_R, OUTC), jnp.float32),
        mesh=mesh,
        scratch_types=[
            pltpu.VMEM((NBUF, 2, C), jnp.int32),
            pltpu.VMEM((NBUF, C, OUTC), jnp.float32),
            pltpu.VMEM_SHARED((ACC_R, OUTC), jnp.float32),
            [pltpu.SemaphoreType.DMA] * NBUF,
            [pltpu.SemaphoreType.DMA] * NBUF,
            [pltpu.SemaphoreType.DMA] * NBUF,
        ],
    )
    def sc_kernel(y_hbm, eidx_hbm, init_hbm, out_hbm,
                  eidx_v, rows_v, acc_sh, semi, semg, sems):
        cid = lax.axis_index("c")
        sid = lax.axis_index("s")
        base = jnp.where(cid == 0, sid * nch0, NS * nch0 + sid * nch1)
        nch = jnp.where(cid == 0, nch0, nch1)
        ngrp = nch // NBUF

        acc_slice = acc_sh.at[pl.ds(sid * ROWS_PER_TILE, ROWS_PER_TILE)]
        if full_init:
            pltpu.sync_copy(
                init_hbm.at[cid, pl.ds(sid * ROWS_PER_TILE, ROWS_PER_TILE)],
                acc_slice)
        else:
            pltpu.sync_copy(init_hbm, acc_slice)
        plsc.subcore_barrier()

        def fire_idx(j, b):
            pltpu.async_copy(eidx_hbm.at[base + j], eidx_v.at[b], semi[b])

        def wait_idx(j, b):
            pltpu.make_async_copy(eidx_hbm.at[base + j], eidx_v.at[b], semi[b]).wait()

        def fire_gather(j, b):
            pltpu.async_copy(y_hbm.at[eidx_v.at[b, 0]], rows_v.at[b], semg[b])

        def wait_gather(j, b):
            pltpu.make_async_copy(y_hbm.at[eidx_v.at[b, 0]], rows_v.at[b], semg[b]).wait()

        def scatter_and_wait(j, b):
            pltpu.async_copy(
                rows_v.at[b], acc_sh.at[eidx_v.at[b, 1]], sems[b], add=True
            ).wait()

        fire_idx(0, 0)
        fire_idx(1, 1)
        wait_idx(0, 0)
        fire_gather(0, 0)

        def step(j, b, fire_next_idx, fire_next_gather):
            bn = 1 - b
            if fire_next_gather:
                wait_idx(j + 1, bn)
                fire_gather(j + 1, bn)
            wait_gather(j, b)
            scatter_and_wait(j, b)
            if fire_next_idx:
                fire_idx(j + 2, b)

        def group(g, carry):
            j = g * 2
            step(j, 0, True, True)
            step(j + 1, 1, True, True)
            return carry

        lax.fori_loop(0, ngrp - 1, group, 0)
        j = (ngrp - 1) * 2
        step(j, 0, False, True)
        step(j + 1, 1, False, False)

        plsc.subcore_barrier()
        pltpu.sync_copy(acc_sh.at[pl.ds(sid * ROWS_PER_TILE, ROWS_PER_TILE)],
                        out_hbm.at[cid, pl.ds(sid * ROWS_PER_TILE, ROWS_PER_TILE)])

    return sc_kernel(y2, eidx, init)


def _edge_chunks(src, dst, k0, k1, nch0, nch1):
    nq = NS * (nch0 + nch1)
    ne = (k1 - k0) * EK
    gidx = (src[k0 * EK:k1 * EK].reshape(k1 - k0, EK)
            + (N * jnp.arange(k1 - k0, dtype=jnp.int32))[:, None]).reshape(ne)
    dstp = dst[k0 * EK:k1 * EK]
    pad = nq * C - ne
    gidx = jnp.concatenate([gidx, jnp.zeros((pad,), jnp.int32)])
    dump = N + (jnp.arange(pad, dtype=jnp.int32) % (ACC_R - N))
    dstp = jnp.concatenate([dstp, dump])
    return jnp.stack([gidx.reshape(nq, C), dstp.reshape(nq, C)], axis=1)


def _bn_body(p_ref, g_ref, b_ref, o_ref):
    s = p_ref[0, :N, :] + p_ref[1, :N, :]
    mean = jnp.mean(s, axis=0, keepdims=True)
    var = jnp.mean((s - mean) ** 2, axis=0, keepdims=True)
    o_ref[...] = jnp.maximum(
        (s - mean) * lax.rsqrt(var + 1e-5) * g_ref[...] + b_ref[...], 0.0
    )


def _tc_bn_relu(parts, gamma, beta):
    return pl.pallas_call(
        _bn_body,
        out_shape=jax.ShapeDtypeStruct((N, OUTC), jnp.float32),
    )(parts, gamma.reshape(1, OUTC), beta.reshape(1, OUTC))


def kernel(x, W, gamma, beta, edge_index):
    src = edge_index[0]
    dst = edge_index[1]

    eidx_a = _edge_chunks(src, dst, 0, KA, NCH0A, NCH1A)
    eidx_b = _edge_chunks(src, dst, KA, K, NCH0B, NCH1B)
    zeros = jnp.zeros((ROWS_PER_TILE, OUTC), jnp.float32)

    y_a = _tc_matmul(x, W[:KA])
    parts_a = _sc_accum(y_a, eidx_a, zeros, NCH0A, NCH1A)
    y_b = _tc_matmul(x, W[KA:])
    parts_b = _sc_accum(y_b, eidx_b, parts_a, NCH0B, NCH1B)
    return _tc_bn_relu(parts_b, gamma, beta)


# speedup vs baseline: 1.4368x; 1.4368x over previous
import functools

import jax
import jax.numpy as jnp
from jax import lax
from jax.experimental import pallas as pl
from jax.experimental.pallas import tpu as pltpu
from jax.experimental.pallas import tpu_sc as plsc

N = 10000
E = 324000
K = 27
EK = E // K
INC = 128
OUTC = 128

NC = 2
NS = 16
NW = NC * NS

C = 128
NBUF = 2

ROWS_PER_TILE = 632
ACC_R = NS * ROWS_PER_TILE

NB = 5
BLK = N // NB

NCH0, NCH1 = 128, 32


def _matmul_body(x_ref, w_ref, y_ref):
    y_ref[...] = jnp.dot(
        x_ref[...], w_ref[0], preferred_element_type=jnp.float32
    )[None]


def _tc_matmul(x, Wp):
    kspan = Wp.shape[0]
    return pl.pallas_call(
        _matmul_body,
        grid=(NB, kspan),
        in_specs=[
            pl.BlockSpec((BLK, INC), lambda nb, k: (nb, 0)),
            pl.BlockSpec((1, INC, OUTC), lambda nb, k: (k, 0, 0)),
        ],
        out_specs=pl.BlockSpec((1, BLK, OUTC), lambda nb, k: (k, nb, 0)),
        out_shape=jax.ShapeDtypeStruct((kspan, N, OUTC), jnp.float32),
    )(x, Wp).reshape(kspan * N, OUTC)


def _sc_accum(y2, eidx, init, nch0, nch1):
    full_init = init.ndim == 3
    mesh = plsc.VectorSubcoreMesh(
        core_axis_name="c", subcore_axis_name="s", num_cores=NC, num_subcores=NS
    )

    @functools.partial(
        pl.kernel,
        out_type=jax.ShapeDtypeStruct((NC, ACC_R, OUTC), jnp.float32),
        mesh=mesh,
        scratch_types=[
            pltpu.VMEM((NBUF, 2, C), jnp.int32),
            pltpu.VMEM((NBUF, C, OUTC), jnp.float32),
            pltpu.VMEM_SHARED((ACC_R, OUTC), jnp.float32),
            [pltpu.SemaphoreType.DMA] * NBUF,
            [pltpu.SemaphoreType.DMA] * NBUF,
            [pltpu.SemaphoreType.DMA] * NBUF,
        ],
    )
    def sc_kernel(y_hbm, eidx_hbm, init_hbm, out_hbm,
                  eidx_v, rows_v, acc_sh, semi, semg, sems):
        cid = lax.axis_index("c")
        sid = lax.axis_index("s")
        base = jnp.where(cid == 0, sid * nch0, NS * nch0 + sid * nch1)
        nch = jnp.where(cid == 0, nch0, nch1)
        ngrp = nch // NBUF

        acc_slice = acc_sh.at[pl.ds(sid * ROWS_PER_TILE, ROWS_PER_TILE)]
        if full_init:
            pltpu.sync_copy(
                init_hbm.at[cid, pl.ds(sid * ROWS_PER_TILE, ROWS_PER_TILE)],
                acc_slice)
        else:
            pltpu.sync_copy(init_hbm, acc_slice)
        plsc.subcore_barrier()

        def fire_idx(j, b):
            pltpu.async_copy(eidx_hbm.at[base + j], eidx_v.at[b], semi[b])

        def wait_idx(j, b):
            pltpu.make_async_copy(eidx_hbm.at[base + j], eidx_v.at[b], semi[b]).wait()

        def fire_gather(j, b):
            pltpu.async_copy(y_hbm.at[eidx_v.at[b, 0]], rows_v.at[b], semg[b])

        def wait_gather(j, b):
            pltpu.make_async_copy(y_hbm.at[eidx_v.at[b, 0]], rows_v.at[b], semg[b]).wait()

        def scatter_and_wait(j, b):
            pltpu.async_copy(
                rows_v.at[b], acc_sh.at[eidx_v.at[b, 1]], sems[b], add=True
            ).wait()

        fire_idx(0, 0)
        fire_idx(1, 1)
        wait_idx(0, 0)
        fire_gather(0, 0)

        def step(j, b, fire_next_idx, fire_next_gather):
            bn = 1 - b
            if fire_next_gather:
                wait_idx(j + 1, bn)
                fire_gather(j + 1, bn)
            wait_gather(j, b)
            scatter_and_wait(j, b)
            if fire_next_idx:
                fire_idx(j + 2, b)

        def group(g, carry):
            j = g * 2
            step(j, 0, True, True)
            step(j + 1, 1, True, True)
            return carry

        lax.fori_loop(0, ngrp - 1, group, 0)
        j = (ngrp - 1) * 2
        step(j, 0, False, True)
        step(j + 1, 1, False, False)

        plsc.subcore_barrier()
        pltpu.sync_copy(acc_sh.at[pl.ds(sid * ROWS_PER_TILE, ROWS_PER_TILE)],
                        out_hbm.at[cid, pl.ds(sid * ROWS_PER_TILE, ROWS_PER_TILE)])

    return sc_kernel(y2, eidx, init)


def _edge_chunks(src, dst, k0, k1, nch0, nch1):
    nq = NS * (nch0 + nch1)
    ne = (k1 - k0) * EK
    gidx = (src[k0 * EK:k1 * EK].reshape(k1 - k0, EK)
            + (N * jnp.arange(k1 - k0, dtype=jnp.int32))[:, None]).reshape(ne)
    dstp = dst[k0 * EK:k1 * EK]
    pad = nq * C - ne
    gidx = jnp.concatenate([gidx, jnp.zeros((pad,), jnp.int32)])
    dump = N + (jnp.arange(pad, dtype=jnp.int32) % (ACC_R - N))
    dstp = jnp.concatenate([dstp, dump])
    return jnp.stack([gidx.reshape(nq, C), dstp.reshape(nq, C)], axis=1)


def _bn_body(p_ref, g_ref, b_ref, o_ref):
    s = p_ref[0, :N, :] + p_ref[1, :N, :]
    mean = jnp.mean(s, axis=0, keepdims=True)
    var = jnp.mean((s - mean) ** 2, axis=0, keepdims=True)
    o_ref[...] = jnp.maximum(
        (s - mean) * lax.rsqrt(var + 1e-5) * g_ref[...] + b_ref[...], 0.0
    )


def _tc_bn_relu(parts, gamma, beta):
    return pl.pallas_call(
        _bn_body,
        out_shape=jax.ShapeDtypeStruct((N, OUTC), jnp.float32),
    )(parts, gamma.reshape(1, OUTC), beta.reshape(1, OUTC))


def kernel(x, W, gamma, beta, edge_index):
    src = edge_index[0]
    dst = edge_index[1]

    eidx = _edge_chunks(src, dst, 0, K, NCH0, NCH1)
    zeros = jnp.zeros((ROWS_PER_TILE, OUTC), jnp.float32)

    y = _tc_matmul(x, W)
    parts = _sc_accum(y, eidx, zeros, NCH0, NCH1)
    return _tc_bn_relu(parts, gamma, beta)


# speedup vs baseline: 1.4537x; 1.0118x over previous
import functools

import jax
import jax.numpy as jnp
from jax import lax
from jax.experimental import pallas as pl
from jax.experimental.pallas import tpu as pltpu
from jax.experimental.pallas import tpu_sc as plsc

N = 10000
E = 324000
K = 27
EK = E // K
INC = 128
OUTC = 128

NC = 2
NS = 16
NW = NC * NS

C = 128
NBUF = 2

ROWS_PER_TILE = 632
ACC_R = NS * ROWS_PER_TILE

NB = 5
BLK = N // NB

NCH0, NCH1 = 132, 28


def _matmul_body(x_ref, w_ref, y_ref):
    y_ref[...] = jnp.dot(
        x_ref[...], w_ref[0], preferred_element_type=jnp.float32
    )[None]


def _tc_matmul(x, Wp):
    kspan = Wp.shape[0]
    return pl.pallas_call(
        _matmul_body,
        grid=(NB, kspan),
        in_specs=[
            pl.BlockSpec((BLK, INC), lambda nb, k: (nb, 0)),
            pl.BlockSpec((1, INC, OUTC), lambda nb, k: (k, 0, 0)),
        ],
        out_specs=pl.BlockSpec((1, BLK, OUTC), lambda nb, k: (k, nb, 0)),
        out_shape=jax.ShapeDtypeStruct((kspan, N, OUTC), jnp.float32),
    )(x, Wp).reshape(kspan * N, OUTC)


def _sc_accum(y2, eidx, init, nch0, nch1):
    full_init = init.ndim == 3
    mesh = plsc.VectorSubcoreMesh(
        core_axis_name="c", subcore_axis_name="s", num_cores=NC, num_subcores=NS
    )

    @functools.partial(
        pl.kernel,
        out_type=jax.ShapeDtypeStruct((NC, ACC_R, OUTC), jnp.float32),
        mesh=mesh,
        scratch_types=[
            pltpu.VMEM((NBUF, 2, C), jnp.int32),
            pltpu.VMEM((NBUF, C, OUTC), jnp.float32),
            pltpu.VMEM_SHARED((ACC_R, OUTC), jnp.float32),
            [pltpu.SemaphoreType.DMA] * NBUF,
            [pltpu.SemaphoreType.DMA] * NBUF,
            [pltpu.SemaphoreType.DMA] * NBUF,
        ],
    )
    def sc_kernel(y_hbm, eidx_hbm, init_hbm, out_hbm,
                  eidx_v, rows_v, acc_sh, semi, semg, sems):
        cid = lax.axis_index("c")
        sid = lax.axis_index("s")
        base = jnp.where(cid == 0, sid * nch0, NS * nch0 + sid * nch1)
        nch = jnp.where(cid == 0, nch0, nch1)
        ngrp = nch // NBUF

        acc_slice = acc_sh.at[pl.ds(sid * ROWS_PER_TILE, ROWS_PER_TILE)]
        if full_init:
            pltpu.sync_copy(
                init_hbm.at[cid, pl.ds(sid * ROWS_PER_TILE, ROWS_PER_TILE)],
                acc_slice)
        else:
            pltpu.sync_copy(init_hbm, acc_slice)
        plsc.subcore_barrier()

        def fire_idx(j, b):
            pltpu.async_copy(eidx_hbm.at[base + j], eidx_v.at[b], semi[b])

        def wait_idx(j, b):
            pltpu.make_async_copy(eidx_hbm.at[base + j], eidx_v.at[b], semi[b]).wait()

        def fire_gather(j, b):
            pltpu.async_copy(y_hbm.at[eidx_v.at[b, 0]], rows_v.at[b], semg[b])

        def wait_gather(j, b):
            pltpu.make_async_copy(y_hbm.at[eidx_v.at[b, 0]], rows_v.at[b], semg[b]).wait()

        def scatter_and_wait(j, b):
            pltpu.async_copy(
                rows_v.at[b], acc_sh.at[eidx_v.at[b, 1]], sems[b], add=True
            ).wait()

        fire_idx(0, 0)
        fire_idx(1, 1)
        wait_idx(0, 0)
        fire_gather(0, 0)

        def step(j, b, fire_next_idx, fire_next_gather):
            bn = 1 - b
            if fire_next_gather:
                wait_idx(j + 1, bn)
                fire_gather(j + 1, bn)
            wait_gather(j, b)
            scatter_and_wait(j, b)
            if fire_next_idx:
                fire_idx(j + 2, b)

        def group(g, carry):
            j = g * 2
            step(j, 0, True, True)
            step(j + 1, 1, True, True)
            return carry

        lax.fori_loop(0, ngrp - 1, group, 0)
        j = (ngrp - 1) * 2
        step(j, 0, False, True)
        step(j + 1, 1, False, False)

        plsc.subcore_barrier()
        pltpu.sync_copy(acc_sh.at[pl.ds(sid * ROWS_PER_TILE, ROWS_PER_TILE)],
                        out_hbm.at[cid, pl.ds(sid * ROWS_PER_TILE, ROWS_PER_TILE)])

    return sc_kernel(y2, eidx, init)


def _edge_chunks(src, dst, k0, k1, nch0, nch1):
    nq = NS * (nch0 + nch1)
    ne = (k1 - k0) * EK
    gidx = (src[k0 * EK:k1 * EK].reshape(k1 - k0, EK)
            + (N * jnp.arange(k1 - k0, dtype=jnp.int32))[:, None]).reshape(ne)
    dstp = dst[k0 * EK:k1 * EK]
    pad = nq * C - ne
    gidx = jnp.concatenate([gidx, jnp.zeros((pad,), jnp.int32)])
    dump = N + (jnp.arange(pad, dtype=jnp.int32) % (ACC_R - N))
    dstp = jnp.concatenate([dstp, dump])
    return jnp.stack([gidx.reshape(nq, C), dstp.reshape(nq, C)], axis=1)


def _bn_body(p_ref, g_ref, b_ref, o_ref):
    s = p_ref[0, :N, :] + p_ref[1, :N, :]
    mean = jnp.mean(s, axis=0, keepdims=True)
    var = jnp.mean((s - mean) ** 2, axis=0, keepdims=True)
    o_ref[...] = jnp.maximum(
        (s - mean) * lax.rsqrt(var + 1e-5) * g_ref[...] + b_ref[...], 0.0
    )


def _tc_bn_relu(parts, gamma, beta):
    return pl.pallas_call(
        _bn_body,
        out_shape=jax.ShapeDtypeStruct((N, OUTC), jnp.float32),
    )(parts, gamma.reshape(1, OUTC), beta.reshape(1, OUTC))


def kernel(x, W, gamma, beta, edge_index):
    src = edge_index[0]
    dst = edge_index[1]

    eidx = _edge_chunks(src, dst, 0, K, NCH0, NCH1)
    zeros = jnp.zeros((ROWS_PER_TILE, OUTC), jnp.float32)

    y = _tc_matmul(x, W)
    parts = _sc_accum(y, eidx, zeros, NCH0, NCH1)
    return _tc_bn_relu(parts, gamma, beta)


# speedup vs baseline: 1.4554x; 1.0012x over previous
import functools

import jax
import jax.numpy as jnp
from jax import lax
from jax.experimental import pallas as pl
from jax.experimental.pallas import tpu as pltpu
from jax.experimental.pallas import tpu_sc as plsc

N = 10000
E = 324000
K = 27
EK = E // K
INC = 128
OUTC = 128

NC = 2
NS = 16
NW = NC * NS

C = 128
NBUF = 2

ROWS_PER_TILE = 632
ACC_R = NS * ROWS_PER_TILE

NB = 5
BLK = N // NB

NCH0, NCH1 = 132, 28


def _matmul_body(x_ref, w_ref, y_ref):
    y_ref[...] = jnp.dot(
        x_ref[...], w_ref[0], preferred_element_type=jnp.float32
    )[None]


def _tc_matmul(x, Wp):
    kspan = Wp.shape[0]
    return pl.pallas_call(
        _matmul_body,
        grid=(NB, kspan),
        in_specs=[
            pl.BlockSpec((BLK, INC), lambda nb, k: (nb, 0)),
            pl.BlockSpec((1, INC, OUTC), lambda nb, k: (k, 0, 0)),
        ],
        out_specs=pl.BlockSpec((1, BLK, OUTC), lambda nb, k: (k, nb, 0)),
        out_shape=jax.ShapeDtypeStruct((kspan, N, OUTC), jnp.float32),
    )(x, Wp).reshape(kspan * N, OUTC)


def _sc_accum(y2, gidx, dst, init, nch0, nch1):
    full_init = init.ndim == 3
    mesh = plsc.VectorSubcoreMesh(
        core_axis_name="c", subcore_axis_name="s", num_cores=NC, num_subcores=NS
    )

    @functools.partial(
        pl.kernel,
        out_type=jax.ShapeDtypeStruct((NC, ACC_R, OUTC), jnp.float32),
        mesh=mesh,
        scratch_types=[
            pltpu.VMEM((NBUF, 2, C), jnp.int32),
            pltpu.VMEM((NBUF, C, OUTC), jnp.float32),
            pltpu.VMEM_SHARED((ACC_R, OUTC), jnp.float32),
            [pltpu.SemaphoreType.DMA] * NBUF,
            [pltpu.SemaphoreType.DMA] * NBUF,
            [pltpu.SemaphoreType.DMA] * NBUF,
        ],
    )
    def sc_kernel(y_hbm, gidx_hbm, dst_hbm, init_hbm, out_hbm,
                  eidx_v, rows_v, acc_sh, semi, semg, sems):
        cid = lax.axis_index("c")
        sid = lax.axis_index("s")
        base = jnp.where(cid == 0, sid * nch0, NS * nch0 + sid * nch1)
        nch = jnp.where(cid == 0, nch0, nch1)
        ngrp = nch // NBUF

        acc_slice = acc_sh.at[pl.ds(sid * ROWS_PER_TILE, ROWS_PER_TILE)]
        if full_init:
            pltpu.sync_copy(
                init_hbm.at[cid, pl.ds(sid * ROWS_PER_TILE, ROWS_PER_TILE)],
                acc_slice)
        else:
            pltpu.sync_copy(init_hbm, acc_slice)
        plsc.subcore_barrier()

        def fire_idx(j, b):
            pltpu.async_copy(gidx_hbm.at[base + j], eidx_v.at[b, 0], semi[b])
            pltpu.async_copy(dst_hbm.at[base + j], eidx_v.at[b, 1], semi[b])

        def wait_idx(j, b):
            pltpu.make_async_copy(gidx_hbm.at[base + j], eidx_v.at[b, 0], semi[b]).wait()
            pltpu.make_async_copy(dst_hbm.at[base + j], eidx_v.at[b, 1], semi[b]).wait()

        def fire_gather(j, b):
            pltpu.async_copy(y_hbm.at[eidx_v.at[b, 0]], rows_v.at[b], semg[b])

        def wait_gather(j, b):
            pltpu.make_async_copy(y_hbm.at[eidx_v.at[b, 0]], rows_v.at[b], semg[b]).wait()

        def scatter_and_wait(j, b):
            pltpu.async_copy(
                rows_v.at[b], acc_sh.at[eidx_v.at[b, 1]], sems[b], add=True
            ).wait()

        fire_idx(0, 0)
        fire_idx(1, 1)
        wait_idx(0, 0)
        fire_gather(0, 0)

        def step(j, b, fire_next_idx, fire_next_gather):
            bn = 1 - b
            if fire_next_gather:
                wait_idx(j + 1, bn)
                fire_gather(j + 1, bn)
            wait_gather(j, b)
            scatter_and_wait(j, b)
            if fire_next_idx:
                fire_idx(j + 2, b)

        def group(g, carry):
            j = g * 2
            step(j, 0, True, True)
            step(j + 1, 1, True, True)
            return carry

        lax.fori_loop(0, ngrp - 1, group, 0)
        j = (ngrp - 1) * 2
        step(j, 0, False, True)
        step(j + 1, 1, False, False)

        plsc.subcore_barrier()
        pltpu.sync_copy(acc_sh.at[pl.ds(sid * ROWS_PER_TILE, ROWS_PER_TILE)],
                        out_hbm.at[cid, pl.ds(sid * ROWS_PER_TILE, ROWS_PER_TILE)])

    return sc_kernel(y2, gidx, dst, init)


def _edge_chunks(src, dst, k0, k1, nch0, nch1):
    nq = NS * (nch0 + nch1)
    ne = (k1 - k0) * EK
    gidx = (src[k0 * EK:k1 * EK].reshape(k1 - k0, EK)
            + (N * jnp.arange(k1 - k0, dtype=jnp.int32))[:, None]).reshape(ne)
    dstp = dst[k0 * EK:k1 * EK]
    pad = nq * C - ne
    gidx = jnp.concatenate([gidx, jnp.zeros((pad,), jnp.int32)])
    dump = N + (jnp.arange(pad, dtype=jnp.int32) % (ACC_R - N))
    dstp = jnp.concatenate([dstp, dump])
    return gidx.reshape(nq, C), dstp.reshape(nq, C)


def _bn_body(p_ref, g_ref, b_ref, o_ref):
    s = p_ref[0, :N, :] + p_ref[1, :N, :]
    mean = jnp.mean(s, axis=0, keepdims=True)
    var = jnp.mean((s - mean) ** 2, axis=0, keepdims=True)
    o_ref[...] = jnp.maximum(
        (s - mean) * lax.rsqrt(var + 1e-5) * g_ref[...] + b_ref[...], 0.0
    )


def _tc_bn_relu(parts, gamma, beta):
    return pl.pallas_call(
        _bn_body,
        out_shape=jax.ShapeDtypeStruct((N, OUTC), jnp.float32),
    )(parts, gamma.reshape(1, OUTC), beta.reshape(1, OUTC))


def kernel(x, W, gamma, beta, edge_index):
    src = edge_index[0]
    dst = edge_index[1]

    gidx, dstq = _edge_chunks(src, dst, 0, K, NCH0, NCH1)
    zeros = jnp.zeros((ROWS_PER_TILE, OUTC), jnp.float32)

    y = _tc_matmul(x, W)
    parts = _sc_accum(y, gidx, dstq, zeros, NCH0, NCH1)
    return _tc_bn_relu(parts, gamma, beta)
